# Initial kernel scaffold; baseline (speedup 1.0000x reference)
import jax
import jax.numpy as jnp
from jax.experimental import pallas as pl


def kernel(continuous, categorical, W1, b1, ln_g, ln_b, W2, b2, type_embed, cat_tables, cat_proj_W, cat_proj_b):
    raise NotImplementedError("write your pallas kernel here")


# speedup vs baseline: 5.9051x; 5.9051x over previous
import functools

import jax
import jax.numpy as jnp
from jax import lax
from jax.experimental import pallas as pl
from jax.experimental.pallas import tpu as pltpu
from jax.experimental.pallas import tpu_sc as plsc

_B = 16384
_NUM_CONT = 13
_NUM_CAT = 26
_VOCAB = 100000
_EMBED_DIM = 16
_DIM = 128

_NC = 2
_NS = 16
_NW = _NC * _NS

_TOTAL = _B * _NUM_CAT
_PER_W = _TOTAL // _NW
_CHUNK = 128
_CH_PER_W = _PER_W // _CHUNK
_HALVES = 2
_CH_HALF = _CH_PER_W // _HALVES
_ROWS_HALF = _CH_HALF * _CHUNK


def _sc_gather(tables_flat, idx2d):
    mesh = plsc.VectorSubcoreMesh(
        core_axis_name="c", subcore_axis_name="s",
        num_cores=_NC, num_subcores=_NS)

    @functools.partial(
        pl.kernel,
        out_type=jax.ShapeDtypeStruct((_TOTAL, _EMBED_DIM), jnp.float32),
        mesh=mesh,
        scratch_types=[
            pltpu.VMEM((_CH_PER_W, _CHUNK), jnp.int32),
            pltpu.VMEM((_ROWS_HALF, _EMBED_DIM), jnp.float32),
            pltpu.SemaphoreType.DMA,
        ],
        compiler_params=pltpu.CompilerParams(use_tc_tiling_on_sc=False),
    )
    def gather_kernel(tables_hbm, idx_hbm, out_hbm, idx_v, rows_v, sem):
        wid = lax.axis_index("s") * _NC + lax.axis_index("c")
        pltpu.sync_copy(idx_hbm.at[pl.ds(wid * _CH_PER_W, _CH_PER_W)], idx_v)
        for h in range(_HALVES):
            @pl.loop(0, _CH_HALF)
            def _fire(j):
                pltpu.async_copy(
                    tables_hbm.at[idx_v.at[h * _CH_HALF + j]],
                    rows_v.at[pl.ds(j * _CHUNK, _CHUNK)],
                    sem)
            out_slice = out_hbm.at[
                pl.ds(wid * _PER_W + h * _ROWS_HALF, _ROWS_HALF)]
            pltpu.make_async_copy(out_slice, rows_v, sem).wait()
            pltpu.sync_copy(rows_v, out_slice)

    return gather_kernel(tables_flat, idx2d)


_G1 = 16
_G2 = _NUM_CAT - _G1
_K1 = _G1 * _EMBED_DIM
_K2 = _G2 * _EMBED_DIM
_N1 = _G1 * _DIM
_N2 = _G2 * _DIM
_NOUT = (_NUM_CAT + 1) * _DIM
_BBLK = 512


def _tc_body(cont_ref, g_ref, w1_ref, b1_ref, lng_ref, lnb_ref, w2_ref,
             wbd1_ref, wbd2_ref, bcont_ref, bcat_ref, out_ref):
    x = cont_ref[...]
    h = jnp.dot(x, w1_ref[...], preferred_element_type=jnp.float32)
    h = h + b1_ref[...]
    mu = jnp.mean(h, axis=-1, keepdims=True)
    d = h - mu
    var = jnp.mean(d * d, axis=-1, keepdims=True)
    h = d * lax.rsqrt(var + 1e-5) * lng_ref[...] + lnb_ref[...]
    h = h * 0.5 * (1.0 + lax.erf(h * 0.7071067811865476))
    out0 = jnp.dot(h, w2_ref[...], preferred_element_type=jnp.float32)
    g = g_ref[...]
    y1 = jnp.dot(g[:, :_K1], wbd1_ref[...], preferred_element_type=jnp.float32)
    y2 = jnp.dot(g[:, _K1:], wbd2_ref[...], preferred_element_type=jnp.float32)
    out_ref[:, 0:_DIM] = out0 + bcont_ref[...]
    out_ref[:, _DIM:_DIM + _N1] = y1 + bcat_ref[:, :_N1]
    out_ref[:, _DIM + _N1:_NOUT] = y2 + bcat_ref[:, _N1:]


def _tc_fused(continuous, g2d, W1, b1, ln_g, ln_b, W2, Wbd1, Wbd2,
              bias_cont, bias_cat):
    nb = _B // _BBLK
    rep = lambda i: (0, 0)
    return pl.pallas_call(
        _tc_body,
        grid=(nb,),
        in_specs=[
            pl.BlockSpec((_BBLK, _NUM_CONT), lambda i: (i, 0)),
            pl.BlockSpec((_BBLK, _NUM_CAT * _EMBED_DIM), lambda i: (i, 0)),
            pl.BlockSpec((_NUM_CONT, 2 * _DIM), rep),
            pl.BlockSpec((1, 2 * _DIM), rep),
            pl.BlockSpec((1, 2 * _DIM), rep),
            pl.BlockSpec((1, 2 * _DIM), rep),
            pl.BlockSpec((2 * _DIM, _DIM), rep),
            pl.BlockSpec((_K1, _N1), rep),
            pl.BlockSpec((_K2, _N2), rep),
            pl.BlockSpec((1, _DIM), rep),
            pl.BlockSpec((1, _N1 + _N2), rep),
        ],
        out_specs=pl.BlockSpec((_BBLK, _NOUT), lambda i: (i, 0)),
        out_shape=jax.ShapeDtypeStruct((_B, _NOUT), jnp.float32),
        compiler_params=pltpu.CompilerParams(
            dimension_semantics=("parallel",)),
    )(continuous, g2d, W1, b1.reshape(1, -1), ln_g.reshape(1, -1),
      ln_b.reshape(1, -1), W2, Wbd1, Wbd2, bias_cont, bias_cat)


def _block_diag(Wg):
    nf, E, D = Wg.shape
    eye = jnp.eye(nf, dtype=Wg.dtype)
    return (eye[:, None, :, None] * Wg[:, :, None, :]).reshape(nf * E, nf * D)


def kernel(continuous, categorical, W1, b1, ln_g, ln_b, W2, b2, type_embed,
           cat_tables, cat_proj_W, cat_proj_b):
    tables_flat = cat_tables.reshape(_NUM_CAT * _VOCAB, _EMBED_DIM)
    offs = (jnp.arange(_NUM_CAT, dtype=jnp.int32) * _VOCAB)[None, :]
    idx2d = (categorical.astype(jnp.int32) + offs).reshape(
        _TOTAL // _CHUNK, _CHUNK)
    gathered = _sc_gather(tables_flat, idx2d)
    g2d = gathered.reshape(_B, _NUM_CAT * _EMBED_DIM)

    Wbd1 = _block_diag(cat_proj_W[:_G1])
    Wbd2 = _block_diag(cat_proj_W[_G1:])
    bias_cat = (cat_proj_b + type_embed[1][None, :]).reshape(1, _NUM_CAT * _DIM)
    bias_cont = (b2 + type_embed[0]).reshape(1, _DIM)

    out2d = _tc_fused(continuous, g2d, W1, b1, ln_g, ln_b, W2, Wbd1, Wbd2,
                      bias_cont, bias_cat)
    return out2d.reshape(_B, _NUM_CAT + 1, _DIM)


# speedup vs baseline: 7.6146x; 1.2895x over previous
import functools

import jax
import jax.numpy as jnp
from jax import lax
from jax.experimental import pallas as pl
from jax.experimental.pallas import tpu as pltpu
from jax.experimental.pallas import tpu_sc as plsc

_B = 16384
_NUM_CONT = 13
_NUM_CAT = 26
_VOCAB = 100000
_EMBED_DIM = 16
_DIM = 128

_NC = 2
_NS = 16
_NW = _NC * _NS

_TOTAL = _B * _NUM_CAT
_PER_W = _TOTAL // _NW
_CHUNK = 128
_CH_PER_W = _PER_W // _CHUNK
_HALVES = 2
_CH_HALF = _CH_PER_W // _HALVES
_ROWS_HALF = _CH_HALF * _CHUNK


def _sc_gather(tables, idx2d):
    mesh = plsc.VectorSubcoreMesh(
        core_axis_name="c", subcore_axis_name="s",
        num_cores=_NC, num_subcores=_NS)

    @functools.partial(
        pl.kernel,
        out_type=jax.ShapeDtypeStruct((_B, _NUM_CAT * _EMBED_DIM), jnp.float32),
        mesh=mesh,
        scratch_types=[
            pltpu.VMEM((_CH_PER_W, _CHUNK), jnp.int32),
            pltpu.VMEM((_ROWS_HALF, _EMBED_DIM), jnp.float32),
            pltpu.SemaphoreType.DMA,
            pltpu.SemaphoreType.DMA,
        ],
        compiler_params=pltpu.CompilerParams(use_tc_tiling_on_sc=False),
    )
    def gather_kernel(tables_hbm, idx_hbm, out_hbm, idx_v, rows_v, gsem, osem):
        wid = lax.axis_index("s") * _NC + lax.axis_index("c")
        k0 = wid * _CH_PER_W
        pltpu.sync_copy(idx_hbm.at[pl.ds(k0, _CH_PER_W)], idx_v)
        drain_src = tables_hbm.at[0, pl.ds(0, _ROWS_HALF), :]
        for h in range(_HALVES):
            @pl.loop(0, _CH_HALF)
            def _fire(j):
                k = k0 + h * _CH_HALF + j
                pltpu.async_copy(
                    tables_hbm.at[k // 128].at[idx_v.at[h * _CH_HALF + j]],
                    rows_v.at[pl.ds(j * _CHUNK, _CHUNK)],
                    gsem)
            pltpu.make_async_copy(drain_src, rows_v, gsem).wait()

            @pl.loop(0, _CH_HALF)
            def _scatter(j):
                k = k0 + h * _CH_HALF + j
                b0 = (k % 128) * _CHUNK
                pltpu.async_copy(
                    rows_v.at[pl.ds(j * _CHUNK, _CHUNK)],
                    out_hbm.at[pl.ds(b0, _CHUNK),
                               pl.ds((k // 128) * _EMBED_DIM, _EMBED_DIM)],
                    osem)
            pltpu.make_async_copy(drain_src, rows_v, osem).wait()

    return gather_kernel(tables, idx2d)


_G1 = 16
_G2 = _NUM_CAT - _G1
_K1 = _G1 * _EMBED_DIM
_K2 = _G2 * _EMBED_DIM
_N1 = _G1 * _DIM
_N2 = _G2 * _DIM
_NOUT = (_NUM_CAT + 1) * _DIM
_BBLK = 512


def _tc_body(cont_ref, g_ref, w1_ref, b1_ref, lng_ref, lnb_ref, w2_ref,
             wbd1_ref, wbd2_ref, bcont_ref, bcat_ref, out_ref):
    x = cont_ref[...]
    h = jnp.dot(x, w1_ref[...], preferred_element_type=jnp.float32)
    h = h + b1_ref[...]
    mu = jnp.mean(h, axis=-1, keepdims=True)
    d = h - mu
    var = jnp.mean(d * d, axis=-1, keepdims=True)
    h = d * lax.rsqrt(var + 1e-5) * lng_ref[...] + lnb_ref[...]
    h = h * 0.5 * (1.0 + lax.erf(h * 0.7071067811865476))
    out0 = jnp.dot(h, w2_ref[...], preferred_element_type=jnp.float32)
    g = g_ref[...]
    y1 = jnp.dot(g[:, :_K1], wbd1_ref[...], preferred_element_type=jnp.float32)
    y2 = jnp.dot(g[:, _K1:], wbd2_ref[...], preferred_element_type=jnp.float32)
    y1 = y1 + bcat_ref[:, :_N1]
    y2 = y2 + bcat_ref[:, _N1:]
    out_ref[0, :, :] = out0 + bcont_ref[...]
    for f in range(_G1):
        out_ref[1 + f, :, :] = y1[:, f * _DIM:(f + 1) * _DIM]
    for f in range(_G2):
        out_ref[1 + _G1 + f, :, :] = y2[:, f * _DIM:(f + 1) * _DIM]


def _tc_fused(continuous, g2d, W1, b1, ln_g, ln_b, W2, Wbd1, Wbd2,
              bias_cont, bias_cat):
    nb = _B // _BBLK
    rep = lambda i: (0, 0)
    return pl.pallas_call(
        _tc_body,
        grid=(nb,),
        in_specs=[
            pl.BlockSpec((_BBLK, _NUM_CONT), lambda i: (i, 0)),
            pl.BlockSpec((_BBLK, _NUM_CAT * _EMBED_DIM), lambda i: (i, 0)),
            pl.BlockSpec((_NUM_CONT, 2 * _DIM), rep),
            pl.BlockSpec((1, 2 * _DIM), rep),
            pl.BlockSpec((1, 2 * _DIM), rep),
            pl.BlockSpec((1, 2 * _DIM), rep),
            pl.BlockSpec((2 * _DIM, _DIM), rep),
            pl.BlockSpec((_K1, _N1), rep),
            pl.BlockSpec((_K2, _N2), rep),
            pl.BlockSpec((1, _DIM), rep),
            pl.BlockSpec((1, _N1 + _N2), rep),
        ],
        out_specs=pl.BlockSpec((_NUM_CAT + 1, _BBLK, _DIM), lambda i: (0, i, 0)),
        out_shape=jax.ShapeDtypeStruct((_NUM_CAT + 1, _B, _DIM), jnp.float32),
        compiler_params=pltpu.CompilerParams(
            dimension_semantics=("parallel",)),
    )(continuous, g2d, W1, b1.reshape(1, -1), ln_g.reshape(1, -1),
      ln_b.reshape(1, -1), W2, Wbd1, Wbd2, bias_cont, bias_cat)


def _block_diag(Wg):
    nf, E, D = Wg.shape
    eye = jnp.eye(nf, dtype=Wg.dtype)
    return (eye[:, None, :, None] * Wg[:, :, None, :]).reshape(nf * E, nf * D)


def kernel(continuous, categorical, W1, b1, ln_g, ln_b, W2, b2, type_embed,
           cat_tables, cat_proj_W, cat_proj_b):
    idx2d = categorical.astype(jnp.int32).T.reshape(_TOTAL // _CHUNK, _CHUNK)
    g2d = _sc_gather(cat_tables, idx2d)

    Wbd1 = _block_diag(cat_proj_W[:_G1])
    Wbd2 = _block_diag(cat_proj_W[_G1:])
    bias_cat = (cat_proj_b + type_embed[1][None, :]).reshape(1, _NUM_CAT * _DIM)
    bias_cont = (b2 + type_embed[0]).reshape(1, _DIM)

    out3 = _tc_fused(continuous, g2d, W1, b1, ln_g, ln_b, W2, Wbd1, Wbd2,
                     bias_cont, bias_cat)
    return jnp.transpose(out3, (1, 0, 2))


# speedup vs baseline: 16.0765x; 2.1113x over previous
import functools

import jax
import jax.numpy as jnp
from jax import lax
from jax.experimental import pallas as pl
from jax.experimental.pallas import tpu as pltpu
from jax.experimental.pallas import tpu_sc as plsc

_B = 16384
_NUM_CONT = 13
_NUM_CAT = 26
_VOCAB = 100000
_EMBED_DIM = 16
_DIM = 128

_NC = 2
_NS = 16
_NW = _NC * _NS

_TOTAL = _B * _NUM_CAT
_PER_W = _TOTAL // _NW
_CHUNK = 128
_CH_PER_W = _PER_W // _CHUNK
_HALVES = 2
_CH_HALF = _CH_PER_W // _HALVES
_ROWS_HALF = _CH_HALF * _CHUNK


_NROWS = _NUM_CAT * _EMBED_DIM
_RPW = _NROWS // _NW
_GCH = 2048
_NGCH = _B // _GCH


def _sc_gather_t(tables_t, idx2d):
    mesh = plsc.VectorSubcoreMesh(
        core_axis_name="c", subcore_axis_name="s",
        num_cores=_NC, num_subcores=_NS)

    @functools.partial(
        pl.kernel,
        out_type=jax.ShapeDtypeStruct((_NROWS, _B), jnp.float32),
        mesh=mesh,
        scratch_types=[
            pltpu.VMEM((_VOCAB,), jnp.float32),
            pltpu.VMEM((_B // _CHUNK, _CHUNK), jnp.int32),
            pltpu.VMEM((2, _GCH), jnp.float32),
            pltpu.SemaphoreType.DMA,
        ],
        compiler_params=pltpu.CompilerParams(
            use_tc_tiling_on_sc=False, needs_layout_passes=False),
    )
    def gather_kernel(tab_hbm, idx_hbm, out_hbm, row_v, idx_v, out_v, osem):
        wid = lax.axis_index("s") * _NC + lax.axis_index("c")
        r0 = wid * _RPW
        drain_src = tab_hbm.at[0, pl.ds(0, _GCH)]

        @pl.loop(0, _RPW)
        def _row(j):
            r = r0 + j
            f = r // _EMBED_DIM
            pltpu.sync_copy(tab_hbm.at[r], row_v)
            pltpu.sync_copy(idx_hbm.at[pl.ds(f * 128, _B // _CHUNK)], idx_v)
            for c in range(_NGCH):
                p = c % 2
                if c >= 2:
                    pltpu.make_async_copy(drain_src, out_v.at[p], osem).wait()

                @pl.loop(0, _GCH // 16)
                def _vec(t):
                    iv = idx_v[c * (_GCH // _CHUNK) + t // 8,
                               pl.ds((t % 8) * 16, 16)]
                    out_v[p, pl.ds(t * 16, 16)] = plsc.load_gather(row_v, [iv])
                pltpu.async_copy(
                    out_v.at[p], out_hbm.at[r, pl.ds(c * _GCH, _GCH)], osem)
            pltpu.make_async_copy(drain_src, out_v.at[0], osem).wait()
            pltpu.make_async_copy(drain_src, out_v.at[1], osem).wait()

    return gather_kernel(tables_t, idx2d)


_G1 = 16
_G2 = _NUM_CAT - _G1
_K1 = _G1 * _EMBED_DIM
_K2 = _G2 * _EMBED_DIM
_N1 = _G1 * _DIM
_N2 = _G2 * _DIM
_NOUT = (_NUM_CAT + 1) * _DIM
_BBLK = 512


def _tc_body(cont_ref, g_ref, w1_ref, b1_ref, lng_ref, lnb_ref, w2_ref,
             wbd1_ref, wbd2_ref, bcont_ref, bcat_ref, out_ref):
    x = cont_ref[...]
    h = jnp.dot(x, w1_ref[...], preferred_element_type=jnp.float32)
    h = h + b1_ref[...]
    mu = jnp.mean(h, axis=-1, keepdims=True)
    d = h - mu
    var = jnp.mean(d * d, axis=-1, keepdims=True)
    h = d * lax.rsqrt(var + 1e-5) * lng_ref[...] + lnb_ref[...]
    h = h * 0.5 * (1.0 + lax.erf(h * 0.7071067811865476))
    out0 = jnp.dot(h, w2_ref[...], preferred_element_type=jnp.float32)
    g = g_ref[...]
    dn = (((0,), (0,)), ((), ()))
    y1 = lax.dot_general(g[:_K1, :], wbd1_ref[...], dn,
                         preferred_element_type=jnp.float32)
    y2 = lax.dot_general(g[_K1:, :], wbd2_ref[...], dn,
                         preferred_element_type=jnp.float32)
    y1 = y1 + bcat_ref[:, :_N1]
    y2 = y2 + bcat_ref[:, _N1:]
    out_ref[0, :, :] = out0 + bcont_ref[...]
    for f in range(_G1):
        out_ref[1 + f, :, :] = y1[:, f * _DIM:(f + 1) * _DIM]
    for f in range(_G2):
        out_ref[1 + _G1 + f, :, :] = y2[:, f * _DIM:(f + 1) * _DIM]


def _tc_fused(continuous, g2d, W1, b1, ln_g, ln_b, W2, Wbd1, Wbd2,
              bias_cont, bias_cat):
    nb = _B // _BBLK
    rep = lambda i: (0, 0)
    return pl.pallas_call(
        _tc_body,
        grid=(nb,),
        in_specs=[
            pl.BlockSpec((_BBLK, _NUM_CONT), lambda i: (i, 0)),
            pl.BlockSpec((_NROWS, _BBLK), lambda i: (0, i)),
            pl.BlockSpec((_NUM_CONT, 2 * _DIM), rep),
            pl.BlockSpec((1, 2 * _DIM), rep),
            pl.BlockSpec((1, 2 * _DIM), rep),
            pl.BlockSpec((1, 2 * _DIM), rep),
            pl.BlockSpec((2 * _DIM, _DIM), rep),
            pl.BlockSpec((_K1, _N1), rep),
            pl.BlockSpec((_K2, _N2), rep),
            pl.BlockSpec((1, _DIM), rep),
            pl.BlockSpec((1, _N1 + _N2), rep),
        ],
        out_specs=pl.BlockSpec((_NUM_CAT + 1, _BBLK, _DIM), lambda i: (0, i, 0)),
        out_shape=jax.ShapeDtypeStruct((_NUM_CAT + 1, _B, _DIM), jnp.float32),
        compiler_params=pltpu.CompilerParams(
            dimension_semantics=("parallel",)),
    )(continuous, g2d, W1, b1.reshape(1, -1), ln_g.reshape(1, -1),
      ln_b.reshape(1, -1), W2, Wbd1, Wbd2, bias_cont, bias_cat)


def _block_diag(Wg):
    nf, E, D = Wg.shape
    eye = jnp.eye(nf, dtype=Wg.dtype)
    return (eye[:, None, :, None] * Wg[:, :, None, :]).reshape(nf * E, nf * D)


def kernel(continuous, categorical, W1, b1, ln_g, ln_b, W2, b2, type_embed,
           cat_tables, cat_proj_W, cat_proj_b):
    idx2d = categorical.astype(jnp.int32).T.reshape(_TOTAL // _CHUNK, _CHUNK)
    tt = jnp.transpose(cat_tables, (0, 2, 1)).reshape(_NROWS, _VOCAB)
    gT = _sc_gather_t(tt, idx2d)
    g2d = gT

    Wbd1 = _block_diag(cat_proj_W[:_G1])
    Wbd2 = _block_diag(cat_proj_W[_G1:])
    bias_cat = (cat_proj_b + type_embed[1][None, :]).reshape(1, _NUM_CAT * _DIM)
    bias_cont = (b2 + type_embed[0]).reshape(1, _DIM)

    out3 = _tc_fused(continuous, g2d, W1, b1, ln_g, ln_b, W2, Wbd1, Wbd2,
                     bias_cont, bias_cat)
    return jnp.transpose(out3, (1, 0, 2))


# speedup vs baseline: 18.0772x; 1.1245x over previous
import functools

import jax
import jax.numpy as jnp
from jax import lax
from jax.experimental import pallas as pl
from jax.experimental.pallas import tpu as pltpu
from jax.experimental.pallas import tpu_sc as plsc

_B = 16384
_NUM_CONT = 13
_NUM_CAT = 26
_VOCAB = 100000
_EMBED_DIM = 16
_DIM = 128

_NC = 2
_NS = 16
_NW = _NC * _NS

_TOTAL = _B * _NUM_CAT
_PER_W = _TOTAL // _NW
_CHUNK = 128
_CH_PER_W = _PER_W // _CHUNK
_HALVES = 2
_CH_HALF = _CH_PER_W // _HALVES
_ROWS_HALF = _CH_HALF * _CHUNK


_NROWS = _NUM_CAT * _EMBED_DIM
_RPW = _NROWS // _NW
_GCH = 2048
_NGCH = _B // _GCH


def _sc_gather_t(tables_t, idx2d):
    mesh = plsc.VectorSubcoreMesh(
        core_axis_name="c", subcore_axis_name="s",
        num_cores=_NC, num_subcores=_NS)

    @functools.partial(
        pl.kernel,
        out_type=jax.ShapeDtypeStruct((_NROWS, _B), jnp.float32),
        mesh=mesh,
        scratch_types=[
            pltpu.VMEM((_VOCAB,), jnp.float32),
            pltpu.VMEM((_B // _CHUNK, _CHUNK), jnp.int32),
            pltpu.VMEM((2, _GCH), jnp.float32),
            pltpu.SemaphoreType.DMA,
        ],
        compiler_params=pltpu.CompilerParams(
            use_tc_tiling_on_sc=False, needs_layout_passes=False),
    )
    def gather_kernel(tab_hbm, idx_hbm, out_hbm, row_v, idx_v, out_v, osem):
        wid = lax.axis_index("s") * _NC + lax.axis_index("c")
        r0 = wid * _RPW
        drain_src = tab_hbm.at[0, pl.ds(0, _GCH)]

        @pl.loop(0, _RPW)
        def _row(j):
            r = r0 + j
            f = r // _EMBED_DIM
            pltpu.sync_copy(tab_hbm.at[r], row_v)
            pltpu.sync_copy(idx_hbm.at[pl.ds(f * 128, _B // _CHUNK)], idx_v)
            for c in range(_NGCH):
                p = c % 2
                if c >= 2:
                    pltpu.make_async_copy(drain_src, out_v.at[p], osem).wait()

                @pl.loop(0, _GCH // _CHUNK)
                def _vec(t):
                    row = c * (_GCH // _CHUNK) + t
                    for s in range(_CHUNK // 16):
                        iv = idx_v[row, pl.ds(s * 16, 16)]
                        out_v[p, pl.ds(t * _CHUNK + s * 16, 16)] = (
                            plsc.load_gather(row_v, [iv]))
                pltpu.async_copy(
                    out_v.at[p], out_hbm.at[r, pl.ds(c * _GCH, _GCH)], osem)
            pltpu.make_async_copy(drain_src, out_v.at[0], osem).wait()
            pltpu.make_async_copy(drain_src, out_v.at[1], osem).wait()

    return gather_kernel(tables_t, idx2d)


_G1 = 16
_G2 = _NUM_CAT - _G1
_K1 = _G1 * _EMBED_DIM
_K2 = _G2 * _EMBED_DIM
_N1 = _G1 * _DIM
_N2 = _G2 * _DIM
_NOUT = (_NUM_CAT + 1) * _DIM
_BBLK = 512


def _tc_body(cont_ref, g_ref, w1_ref, b1_ref, lng_ref, lnb_ref, w2_ref,
             wbd1_ref, wbd2_ref, bcont_ref, bcat_ref, out_ref):
    x = cont_ref[...]
    h = jnp.dot(x, w1_ref[...], preferred_element_type=jnp.float32)
    h = h + b1_ref[...]
    mu = jnp.mean(h, axis=-1, keepdims=True)
    d = h - mu
    var = jnp.mean(d * d, axis=-1, keepdims=True)
    h = d * lax.rsqrt(var + 1e-5) * lng_ref[...] + lnb_ref[...]
    h = h * 0.5 * (1.0 + lax.erf(h * 0.7071067811865476))
    out0 = jnp.dot(h, w2_ref[...], preferred_element_type=jnp.float32)
    g = g_ref[...]
    dn = (((0,), (0,)), ((), ()))
    y1 = lax.dot_general(g[:_K1, :], wbd1_ref[...], dn,
                         preferred_element_type=jnp.float32)
    y2 = lax.dot_general(g[_K1:, :], wbd2_ref[...], dn,
                         preferred_element_type=jnp.float32)
    y1 = y1 + bcat_ref[:, :_N1]
    y2 = y2 + bcat_ref[:, _N1:]
    out_ref[0, :, :] = out0 + bcont_ref[...]
    for f in range(_G1):
        out_ref[1 + f, :, :] = y1[:, f * _DIM:(f + 1) * _DIM]
    for f in range(_G2):
        out_ref[1 + _G1 + f, :, :] = y2[:, f * _DIM:(f + 1) * _DIM]


def _tc_fused(continuous, g2d, W1, b1, ln_g, ln_b, W2, Wbd1, Wbd2,
              bias_cont, bias_cat):
    nb = _B // _BBLK
    rep = lambda i: (0, 0)
    return pl.pallas_call(
        _tc_body,
        grid=(nb,),
        in_specs=[
            pl.BlockSpec((_BBLK, _NUM_CONT), lambda i: (i, 0)),
            pl.BlockSpec((_NROWS, _BBLK), lambda i: (0, i)),
            pl.BlockSpec((_NUM_CONT, 2 * _DIM), rep),
            pl.BlockSpec((1, 2 * _DIM), rep),
            pl.BlockSpec((1, 2 * _DIM), rep),
            pl.BlockSpec((1, 2 * _DIM), rep),
            pl.BlockSpec((2 * _DIM, _DIM), rep),
            pl.BlockSpec((_K1, _N1), rep),
            pl.BlockSpec((_K2, _N2), rep),
            pl.BlockSpec((1, _DIM), rep),
            pl.BlockSpec((1, _N1 + _N2), rep),
        ],
        out_specs=pl.BlockSpec((_NUM_CAT + 1, _BBLK, _DIM), lambda i: (0, i, 0)),
        out_shape=jax.ShapeDtypeStruct((_NUM_CAT + 1, _B, _DIM), jnp.float32),
        compiler_params=pltpu.CompilerParams(
            dimension_semantics=("parallel",)),
    )(continuous, g2d, W1, b1.reshape(1, -1), ln_g.reshape(1, -1),
      ln_b.reshape(1, -1), W2, Wbd1, Wbd2, bias_cont, bias_cat)


def _block_diag(Wg):
    nf, E, D = Wg.shape
    eye = jnp.eye(nf, dtype=Wg.dtype)
    return (eye[:, None, :, None] * Wg[:, :, None, :]).reshape(nf * E, nf * D)


def kernel(continuous, categorical, W1, b1, ln_g, ln_b, W2, b2, type_embed,
           cat_tables, cat_proj_W, cat_proj_b):
    idx2d = categorical.astype(jnp.int32).T.reshape(_TOTAL // _CHUNK, _CHUNK)
    tt = jnp.transpose(cat_tables, (0, 2, 1)).reshape(_NROWS, _VOCAB)
    gT = _sc_gather_t(tt, idx2d)
    g2d = gT

    Wbd1 = _block_diag(cat_proj_W[:_G1])
    Wbd2 = _block_diag(cat_proj_W[_G1:])
    bias_cat = (cat_proj_b + type_embed[1][None, :]).reshape(1, _NUM_CAT * _DIM)
    bias_cont = (b2 + type_embed[0]).reshape(1, _DIM)

    out3 = _tc_fused(continuous, g2d, W1, b1, ln_g, ln_b, W2, Wbd1, Wbd2,
                     bias_cont, bias_cat)
    return jnp.transpose(out3, (1, 0, 2))
